# Initial kernel scaffold; baseline (speedup 1.0000x reference)
import jax
import jax.numpy as jnp
from jax.experimental import pallas as pl


def kernel(x, edge_index, W1_l, W1_r, b1, bn1_gamma, bn1_beta, W2_l, W2_r, b2, bn2_gamma, bn2_beta):
    raise NotImplementedError("write your pallas kernel here")


# speedup vs baseline: 11.2479x; 11.2479x over previous
import functools

import jax
import jax.numpy as jnp
from jax import lax
from jax.experimental import pallas as pl
from jax.experimental.pallas import tpu as pltpu
from jax.experimental.pallas import tpu_sc as plsc

_N = 10000
_E = 320000
_D_IN = 128
_D_HID = 32
_D_OUT = 2
_W2P = 16
_EPS = 1e-5

_NC = 2
_NS = 16
_NW = _NC * _NS
_CHUNK = 128
_CPT = 79
_EPT = _CPT * _CHUNK
_E_PAD = _NW * _EPT
_ROWS = 10112
_RPT = _ROWS // _NS
_DW = 8


def _sc_agg(width, with_deg):
  mesh = plsc.VectorSubcoreMesh(core_axis_name="c", subcore_axis_name="s",
                                num_cores=_NC, num_subcores=_NS)
  out_type = [jax.ShapeDtypeStruct((_NC * _ROWS, width), jnp.float32)]
  scratch = [
      pltpu.VMEM((_CPT, _CHUNK), jnp.int32),
      pltpu.VMEM((_CPT, _CHUNK), jnp.int32),
      pltpu.VMEM((_CHUNK, width), jnp.float32),
      pltpu.VMEM_SHARED((_ROWS, width), jnp.float32),
      pltpu.SemaphoreType.DMA,
  ]
  if with_deg:
    out_type.append(jax.ShapeDtypeStruct((_NC * _ROWS, _DW), jnp.float32))
    scratch += [
        pltpu.VMEM((_CHUNK, _DW), jnp.float32),
        pltpu.VMEM_SHARED((_ROWS, _DW), jnp.float32),
    ]

  def body(*refs):
    if with_deg:
      (y_hbm, src_hbm, dst_hbm, zf_hbm, zd_hbm, ones_hbm, acc_out, deg_out,
       src_v, dst_v, msgs_v, acc_sh, sem, ones_v, deg_sh) = refs
    else:
      (y_hbm, src_hbm, dst_hbm, zf_hbm, acc_out,
       src_v, dst_v, msgs_v, acc_sh, sem) = refs

    c = lax.axis_index("c")
    s = lax.axis_index("s")
    wid = c * _NS + s
    row0 = s * _RPT

    pltpu.sync_copy(zf_hbm, acc_sh.at[pl.ds(row0, _RPT)])
    if with_deg:
      pltpu.sync_copy(zd_hbm, deg_sh.at[pl.ds(row0, _RPT)])
      pltpu.sync_copy(ones_hbm, ones_v)
    pltpu.sync_copy(src_hbm.at[wid], src_v)
    pltpu.sync_copy(dst_hbm.at[wid], dst_v)
    plsc.subcore_barrier()

    def step(j, carry):
      pltpu.async_copy(y_hbm.at[src_v.at[j]], msgs_v, sem).wait()
      pltpu.sync_copy(msgs_v, acc_sh.at[dst_v.at[j]], add=True)
      if with_deg:
        pltpu.sync_copy(ones_v, deg_sh.at[dst_v.at[j]], add=True)
      return carry

    lax.fori_loop(0, _CPT, step, 0)

    plsc.subcore_barrier()
    out0 = c * _ROWS + row0
    pltpu.sync_copy(acc_sh.at[pl.ds(row0, _RPT)], acc_out.at[pl.ds(out0, _RPT)])
    if with_deg:
      pltpu.sync_copy(deg_sh.at[pl.ds(row0, _RPT)],
                      deg_out.at[pl.ds(out0, _RPT)])

  return pl.kernel(
      body, out_type=out_type, mesh=mesh, scratch_types=scratch,
      compiler_params=pltpu.CompilerParams(use_tc_tiling_on_sc=False))


def _stage_a(x, W1_l, W1_r, b1):
  def body(x_ref, wl_ref, wr_ref, b_ref, y_ref, z_ref):
    xv = x_ref[...]
    y_ref[...] = jnp.dot(xv, wl_ref[...], preferred_element_type=jnp.float32)
    z_ref[...] = (jnp.dot(xv, wr_ref[...], preferred_element_type=jnp.float32)
                  + b_ref[...])

  return pl.pallas_call(
      body,
      out_shape=[jax.ShapeDtypeStruct((_N, _D_HID), jnp.float32),
                 jax.ShapeDtypeStruct((_N, _D_HID), jnp.float32)],
  )(x, W1_l, W1_r, b1)


def _stage_b(acc1, deg3, z1, gamma, beta, W2lp, W2_r, b2):
  def body(acc_ref, deg_ref, z1_ref, g_ref, be_ref, wl_ref, wr_ref, b2_ref,
           y2_ref, z2_ref):
    sums = acc_ref[0, :_N, :] + acc_ref[1, :_N, :]
    deg = deg_ref[0, :_N, :1] + deg_ref[1, :_N, :1]
    invd = 1.0 / jnp.maximum(deg, 1.0)
    pre = sums * invd + z1_ref[...]
    mu = jnp.mean(pre, axis=0, keepdims=True)
    var = jnp.mean((pre - mu) ** 2, axis=0, keepdims=True)
    h = (pre - mu) * lax.rsqrt(var + _EPS) * g_ref[...] + be_ref[...]
    h = jnp.maximum(h, 0.0)
    y2_ref[...] = jnp.dot(h, wl_ref[...], preferred_element_type=jnp.float32)
    z2_ref[...] = (jnp.dot(h, wr_ref[...], preferred_element_type=jnp.float32)
                   + b2_ref[...])

  return pl.pallas_call(
      body,
      out_shape=[jax.ShapeDtypeStruct((_N, _W2P), jnp.float32),
                 jax.ShapeDtypeStruct((_N, _D_OUT), jnp.float32)],
  )(acc1, deg3, z1, gamma, beta, W2lp, W2_r, b2)


def _stage_c(acc2, deg3, z2, gamma, beta):
  def body(acc_ref, deg_ref, z2_ref, g_ref, be_ref, out_ref):
    sums = acc_ref[0, :_N, :_D_OUT] + acc_ref[1, :_N, :_D_OUT]
    deg = deg_ref[0, :_N, :1] + deg_ref[1, :_N, :1]
    invd = 1.0 / jnp.maximum(deg, 1.0)
    pre = sums * invd + z2_ref[...]
    mu = jnp.mean(pre, axis=0, keepdims=True)
    var = jnp.mean((pre - mu) ** 2, axis=0, keepdims=True)
    h = (pre - mu) * lax.rsqrt(var + _EPS) * g_ref[...] + be_ref[...]
    m = jnp.max(h, axis=1, keepdims=True)
    lse = jnp.log(jnp.sum(jnp.exp(h - m), axis=1, keepdims=True)) + m
    out_ref[...] = h - lse

  return pl.pallas_call(
      body,
      out_shape=jax.ShapeDtypeStruct((_N, _D_OUT), jnp.float32),
  )(acc2, deg3, z2, gamma, beta)


def kernel(x, edge_index, W1_l, W1_r, b1, bn1_gamma, bn1_beta,
           W2_l, W2_r, b2, bn2_gamma, bn2_beta):
  src = edge_index[0]
  dst = edge_index[1]
  pad = _E_PAD - _E
  src3 = jnp.concatenate([src, jnp.zeros((pad,), jnp.int32)]).reshape(
      _NW, _CPT, _CHUNK)
  dst3 = jnp.concatenate([dst, jnp.full((pad,), _N, jnp.int32)]).reshape(
      _NW, _CPT, _CHUNK)
  zf32 = jnp.zeros((_RPT, _D_HID), jnp.float32)
  zf16 = jnp.zeros((_RPT, _W2P), jnp.float32)
  zd = jnp.zeros((_RPT, _DW), jnp.float32)
  ones = jnp.ones((_CHUNK, _DW), jnp.float32)
  W2lp = jnp.pad(W2_l, ((0, 0), (0, _W2P - _D_OUT)))

  y1, z1 = _stage_a(x, W1_l, W1_r, b1)
  acc1, deg = _sc_agg(_D_HID, True)(y1, src3, dst3, zf32, zd, ones)
  acc1 = acc1.reshape(_NC, _ROWS, _D_HID)
  deg3 = deg.reshape(_NC, _ROWS, _DW)
  y2p, z2 = _stage_b(acc1, deg3, z1, bn1_gamma, bn1_beta, W2lp, W2_r, b2)
  (acc2,) = _sc_agg(_W2P, False)(y2p, src3, dst3, zf16)
  acc2 = acc2.reshape(_NC, _ROWS, _W2P)
  return _stage_c(acc2, deg3, z2, bn2_gamma, bn2_beta)


# speedup vs baseline: 12.3324x; 1.0964x over previous
import functools

import jax
import jax.numpy as jnp
from jax import lax
from jax.experimental import pallas as pl
from jax.experimental.pallas import tpu as pltpu
from jax.experimental.pallas import tpu_sc as plsc

_N = 10000
_E = 320000
_D_IN = 128
_D_HID = 32
_D_OUT = 2
_W2P = 16
_EPS = 1e-5

_NC = 2
_NS = 16
_NW = _NC * _NS
_CHUNK = 128
_CPT = 80
_NBUF = 4
_EPT = _CPT * _CHUNK
_E_PAD = _NW * _EPT
_ROWS = 10112
_RPT = _ROWS // _NS
_DW = 8


def _sc_agg(width, with_deg):
  mesh = plsc.VectorSubcoreMesh(core_axis_name="c", subcore_axis_name="s",
                                num_cores=_NC, num_subcores=_NS)
  out_type = [jax.ShapeDtypeStruct((_NC * _ROWS, width), jnp.float32)]
  scratch = [
      pltpu.VMEM((_CPT, _CHUNK), jnp.int32),
      pltpu.VMEM((_CPT, _CHUNK), jnp.int32),
      pltpu.VMEM((_NBUF, _CHUNK, width), jnp.float32),
      pltpu.VMEM_SHARED((_ROWS, width), jnp.float32),
  ]
  if with_deg:
    out_type.append(jax.ShapeDtypeStruct((_NC * _ROWS, _DW), jnp.float32))
    scratch += [
        pltpu.VMEM((_CHUNK, _DW), jnp.float32),
        pltpu.VMEM_SHARED((_ROWS, _DW), jnp.float32),
    ]
  scratch += [pltpu.SemaphoreType.DMA] * (2 * _NBUF)

  def body(*refs):
    if with_deg:
      (y_hbm, src_hbm, dst_hbm, zf_hbm, zd_hbm, ones_hbm, acc_out, deg_out,
       src_v, dst_v, msgs_v, acc_sh, ones_v, deg_sh, *sems) = refs
    else:
      (y_hbm, src_hbm, dst_hbm, zf_hbm, acc_out,
       src_v, dst_v, msgs_v, acc_sh, *sems) = refs
    gsem = sems[:_NBUF]
    ssem = sems[_NBUF:]

    c = lax.axis_index("c")
    s = lax.axis_index("s")
    wid = c * _NS + s
    row0 = s * _RPT

    pltpu.sync_copy(zf_hbm, acc_sh.at[pl.ds(row0, _RPT)])
    if with_deg:
      pltpu.sync_copy(zd_hbm, deg_sh.at[pl.ds(row0, _RPT)])
      pltpu.sync_copy(ones_hbm, ones_v)
    pltpu.sync_copy(src_hbm.at[wid], src_v)
    pltpu.sync_copy(dst_hbm.at[wid], dst_v)
    for b in range(_NBUF):
      pltpu.async_copy(y_hbm.at[src_v.at[b]], msgs_v.at[b], gsem[b])
    plsc.subcore_barrier()

    def step(i, carry):
      for b in range(_NBUF):
        j = i * _NBUF + b
        pltpu.make_async_copy(y_hbm.at[src_v.at[j]], msgs_v.at[b],
                              gsem[b]).wait()
        sd = pltpu.async_copy(msgs_v.at[b], acc_sh.at[dst_v.at[j]], ssem[b],
                              add=True)
        if with_deg:
          pltpu.sync_copy(ones_v, deg_sh.at[dst_v.at[j]], add=True)
        sd.wait()

        @pl.when(j + _NBUF < _CPT)
        def _start_next():
          pltpu.async_copy(y_hbm.at[src_v.at[j + _NBUF]], msgs_v.at[b],
                           gsem[b])
      return carry

    lax.fori_loop(0, _CPT // _NBUF, step, 0)

    plsc.subcore_barrier()
    out0 = c * _ROWS + row0
    pltpu.sync_copy(acc_sh.at[pl.ds(row0, _RPT)], acc_out.at[pl.ds(out0, _RPT)])
    if with_deg:
      pltpu.sync_copy(deg_sh.at[pl.ds(row0, _RPT)],
                      deg_out.at[pl.ds(out0, _RPT)])

  return pl.kernel(
      body, out_type=out_type, mesh=mesh, scratch_types=scratch,
      compiler_params=pltpu.CompilerParams(use_tc_tiling_on_sc=False))


def _stage_a(x, W1_l, W1_r, b1):
  def body(x_ref, wl_ref, wr_ref, b_ref, y_ref, z_ref):
    xv = x_ref[...]
    y_ref[...] = jnp.dot(xv, wl_ref[...], preferred_element_type=jnp.float32)
    z_ref[...] = (jnp.dot(xv, wr_ref[...], preferred_element_type=jnp.float32)
                  + b_ref[...])

  return pl.pallas_call(
      body,
      out_shape=[jax.ShapeDtypeStruct((_N, _D_HID), jnp.float32),
                 jax.ShapeDtypeStruct((_N, _D_HID), jnp.float32)],
  )(x, W1_l, W1_r, b1)


def _stage_b(acc1, deg3, z1, gamma, beta, W2lp, W2_r, b2):
  def body(acc_ref, deg_ref, z1_ref, g_ref, be_ref, wl_ref, wr_ref, b2_ref,
           y2_ref, z2_ref):
    sums = acc_ref[0, :_N, :] + acc_ref[1, :_N, :]
    deg = deg_ref[0, :_N, :1] + deg_ref[1, :_N, :1]
    invd = 1.0 / jnp.maximum(deg, 1.0)
    pre = sums * invd + z1_ref[...]
    mu = jnp.mean(pre, axis=0, keepdims=True)
    var = jnp.mean((pre - mu) ** 2, axis=0, keepdims=True)
    h = (pre - mu) * lax.rsqrt(var + _EPS) * g_ref[...] + be_ref[...]
    h = jnp.maximum(h, 0.0)
    y2_ref[...] = jnp.dot(h, wl_ref[...], preferred_element_type=jnp.float32)
    z2_ref[...] = (jnp.dot(h, wr_ref[...], preferred_element_type=jnp.float32)
                   + b2_ref[...])

  return pl.pallas_call(
      body,
      out_shape=[jax.ShapeDtypeStruct((_N, _W2P), jnp.float32),
                 jax.ShapeDtypeStruct((_N, _D_OUT), jnp.float32)],
  )(acc1, deg3, z1, gamma, beta, W2lp, W2_r, b2)


def _stage_c(acc2, deg3, z2, gamma, beta):
  def body(acc_ref, deg_ref, z2_ref, g_ref, be_ref, out_ref):
    sums = acc_ref[0, :_N, :_D_OUT] + acc_ref[1, :_N, :_D_OUT]
    deg = deg_ref[0, :_N, :1] + deg_ref[1, :_N, :1]
    invd = 1.0 / jnp.maximum(deg, 1.0)
    pre = sums * invd + z2_ref[...]
    mu = jnp.mean(pre, axis=0, keepdims=True)
    var = jnp.mean((pre - mu) ** 2, axis=0, keepdims=True)
    h = (pre - mu) * lax.rsqrt(var + _EPS) * g_ref[...] + be_ref[...]
    m = jnp.max(h, axis=1, keepdims=True)
    lse = jnp.log(jnp.sum(jnp.exp(h - m), axis=1, keepdims=True)) + m
    out_ref[...] = h - lse

  return pl.pallas_call(
      body,
      out_shape=jax.ShapeDtypeStruct((_N, _D_OUT), jnp.float32),
  )(acc2, deg3, z2, gamma, beta)


def kernel(x, edge_index, W1_l, W1_r, b1, bn1_gamma, bn1_beta,
           W2_l, W2_r, b2, bn2_gamma, bn2_beta):
  src = edge_index[0]
  dst = edge_index[1]
  pad = _E_PAD - _E
  src3 = jnp.concatenate([src, jnp.zeros((pad,), jnp.int32)]).reshape(
      _NW, _CPT, _CHUNK)
  dst3 = jnp.concatenate([dst, jnp.full((pad,), _N, jnp.int32)]).reshape(
      _NW, _CPT, _CHUNK)
  zf32 = jnp.zeros((_RPT, _D_HID), jnp.float32)
  zf16 = jnp.zeros((_RPT, _W2P), jnp.float32)
  zd = jnp.zeros((_RPT, _DW), jnp.float32)
  ones = jnp.ones((_CHUNK, _DW), jnp.float32)
  W2lp = jnp.pad(W2_l, ((0, 0), (0, _W2P - _D_OUT)))

  y1, z1 = _stage_a(x, W1_l, W1_r, b1)
  acc1, deg = _sc_agg(_D_HID, True)(y1, src3, dst3, zf32, zd, ones)
  acc1 = acc1.reshape(_NC, _ROWS, _D_HID)
  deg3 = deg.reshape(_NC, _ROWS, _DW)
  y2p, z2 = _stage_b(acc1, deg3, z1, bn1_gamma, bn1_beta, W2lp, W2_r, b2)
  (acc2,) = _sc_agg(_W2P, False)(y2p, src3, dst3, zf16)
  acc2 = acc2.reshape(_NC, _ROWS, _W2P)
  return _stage_c(acc2, deg3, z2, bn2_gamma, bn2_beta)
